# baseline (speedup 1.0000x reference)
import functools

import jax
import jax.numpy as jnp
from jax import lax
from jax.experimental import pallas as pl
from jax.experimental.pallas import tpu as pltpu
from jax.experimental.pallas import tpu_sc as plsc

B = 16384
C = 1000

LAMB = max(5.0, 1500.0 / (1.0 + 0.1 * 1))
INV = 1.0 / (1.0 + LAMB)

_NC, _NS, _L = 2, 16, 16
_NW = _NC * _NS
_BPW = B // _NW


@functools.cache
def _make_sc_gather():
    mesh = plsc.VectorSubcoreMesh(
        core_axis_name="c", subcore_axis_name="s",
        num_cores=_NC, num_subcores=_NS,
    )

    @functools.partial(
        pl.kernel,
        out_type=jax.ShapeDtypeStruct((B,), jnp.float32),
        mesh=mesh,
        scratch_types=[
            pltpu.VMEM((_BPW,), jnp.int32),
            pltpu.VMEM((_BPW,), jnp.int32),
            pltpu.VMEM((_BPW,), jnp.float32),
            pltpu.SemaphoreType.DMA,
        ],
    )
    def _sc_gather(phi_hbm, tgt_hbm, pt_out, tgt_v, fidx_v, pg_v, sem):
        wid = lax.axis_index("s") * _NC + lax.axis_index("c")
        base = wid * _BPW
        pltpu.sync_copy(tgt_hbm.at[pl.ds(base, _BPW)], tgt_v)
        lane = lax.broadcasted_iota(jnp.int32, (_L,), 0)
        for g in range(_BPW // _L):
            t16 = tgt_v[pl.ds(g * _L, _L)]
            fidx_v[pl.ds(g * _L, _L)] = (base + g * _L + lane) * jnp.int32(C) + t16
        pltpu.async_copy(phi_hbm.at[fidx_v], pg_v, sem).wait()
        pltpu.sync_copy(pg_v, pt_out.at[pl.ds(base, _BPW)])

    return _sc_gather


_R = 2048
_NT = B // _R


def _rows_body(cos_ref, tgt_ref, m_ref, s_ref, ct_ref):
    x = cos_ref[...]
    t = tgt_ref[...]
    m = jnp.max(x, axis=1, keepdims=True)
    s = jnp.sum(jnp.exp(x - m), axis=1, keepdims=True)
    col = lax.broadcasted_iota(jnp.int32, x.shape, 1)
    ct = jnp.sum(jnp.where(col == t, x, 0.0), axis=1, keepdims=True)
    m_ref[...] = m.reshape(_R)
    s_ref[...] = s.reshape(_R)
    ct_ref[...] = ct.reshape(_R)


_rows_call = pl.pallas_call(
    _rows_body,
    grid=(_NT,),
    in_specs=[
        pl.BlockSpec((_R, C), lambda i: (i, 0)),
        pl.BlockSpec((_R, 1), lambda i: (i, 0)),
    ],
    out_specs=[
        pl.BlockSpec((_R,), lambda i: (i,)),
        pl.BlockSpec((_R,), lambda i: (i,)),
        pl.BlockSpec((_R,), lambda i: (i,)),
    ],
    out_shape=[
        jax.ShapeDtypeStruct((B,), jnp.float32),
        jax.ShapeDtypeStruct((B,), jnp.float32),
        jax.ShapeDtypeStruct((B,), jnp.float32),
    ],
)


def _combine_body(m_ref, s_ref, ct_ref, pt_ref, out_ref):
    M = m_ref[...]
    S0 = s_ref[...]
    ct = ct_ref[...]
    pt = pt_ref[...]
    out_t = ct - ct * INV + pt * INV
    m = jnp.maximum(M, out_t)
    S = S0 * jnp.exp(M - m) - jnp.exp(ct - m) + jnp.exp(out_t - m)
    loss = m + jnp.log(S) - out_t
    out_ref[...] = (jnp.sum(loss) * (1.0 / B)).reshape(1, 1)


_combine_call = pl.pallas_call(
    _combine_body,
    out_shape=jax.ShapeDtypeStruct((1, 1), jnp.float32),
)


def kernel(cos_theta, phi_theta, target):
    tgt = target.reshape(-1).astype(jnp.int32)
    pt = _make_sc_gather()(phi_theta.reshape(B * C), tgt)
    m, s0, ct = _rows_call(cos_theta, tgt.reshape(B, 1))
    out = _combine_call(m, s0, ct, pt)
    return out.reshape(())


# speedup vs baseline: 1.4862x; 1.4862x over previous
import jax
import jax.numpy as jnp
from jax import lax
from jax.experimental import pallas as pl

B = 16384
C = 1000

LAMB = max(5.0, 1500.0 / (1.0 + 0.1 * 1))
INV = 1.0 / (1.0 + LAMB)

_R = 1024
_NG = B // (2 * _R)


def _half_loss(x, p, t):
    col = lax.broadcasted_iota(jnp.int32, x.shape, 1)
    onehot = col == t
    out = jnp.where(onehot, x - x * INV + p * INV, x)
    m = jnp.max(out, axis=1, keepdims=True)
    s = jnp.sum(jnp.exp(out - m), axis=1, keepdims=True)
    out_t = jnp.sum(jnp.where(onehot, out, 0.0), axis=1, keepdims=True)
    return jnp.sum(m + jnp.log(s) - out_t, axis=0, keepdims=True)


def _body(cosA_ref, cosB_ref, phiA_ref, phiB_ref, tgtA_ref, tgtB_ref, out_ref):
    acc = (_half_loss(cosA_ref[...], phiA_ref[...], tgtA_ref[...])
           + _half_loss(cosB_ref[...], phiB_ref[...], tgtB_ref[...]))

    @pl.when(pl.program_id(0) == 0)
    def _():
        out_ref[...] = jnp.zeros_like(out_ref)

    out_ref[...] += acc * (1.0 / B)


_call = pl.pallas_call(
    _body,
    grid=(_NG,),
    in_specs=[
        pl.BlockSpec((_R, C), lambda i: (i, 0)),
        pl.BlockSpec((_R, C), lambda i: (i + _NG, 0)),
        pl.BlockSpec((_R, C), lambda i: (i, 0)),
        pl.BlockSpec((_R, C), lambda i: (i + _NG, 0)),
        pl.BlockSpec((_R, 1), lambda i: (i, 0)),
        pl.BlockSpec((_R, 1), lambda i: (i + _NG, 0)),
    ],
    out_specs=pl.BlockSpec((1, 1), lambda i: (0, 0)),
    out_shape=jax.ShapeDtypeStruct((1, 1), jnp.float32),
)


def kernel(cos_theta, phi_theta, target):
    tgt = target.reshape(-1).astype(jnp.int32).reshape(B, 1)
    out = _call(cos_theta, cos_theta, phi_theta, phi_theta, tgt, tgt)
    return out.reshape(())
